# baseline (speedup 1.0000x reference)
import jax
import jax.numpy as jnp
from jax.experimental import pallas as pl


def _body(x_ref, rv_ref, wr_ref, br_ref, emb_ref, cw_ref, cb_ref, out_ref):
    E, C_OUT, KC = cw_ref.shape
    C_IN = KC // 9
    HOW = out_ref.shape[2]
    W = (x_ref.shape[2] - HOW - 2) // 2

    rv = rv_ref[0]
    r = jax.lax.dot_general(rv, wr_ref[...],
                            (((1,), (1,)), ((), ())),
                            preferred_element_type=jnp.float32)
    r = r + br_ref[...]

    rn = r / (jnp.sqrt(jnp.sum(r * r, axis=-1, keepdims=True)) + 1e-8)
    emb = emb_ref[...]
    en = emb / (jnp.sqrt(jnp.sum(emb * emb, axis=-1, keepdims=True)) + 1e-8)
    sim = jax.lax.dot_general(en, rn, (((1,), (1,)), ((), ())),
                              preferred_element_type=jnp.float32)

    m = jnp.max(sim, axis=0, keepdims=True)
    ex = jnp.exp(sim - m)
    wts = ex / jnp.sum(ex, axis=0, keepdims=True)
    wts = wts / jnp.sum(wts, axis=0, keepdims=True)

    comb = jnp.sum(cw_ref[...] * wts.reshape(E, 1, 1), axis=0)
    beff = jax.lax.dot_general(cb_ref[...], wts, (((0,), (0,)), ((), ())),
                               preferred_element_type=jnp.float32)

    xv = x_ref[0]
    combb = comb.astype(jnp.bfloat16)
    acc = jnp.zeros((C_OUT, HOW), dtype=jnp.float32)
    for kh in range(3):
        for kw in range(3):
            k = kh * 3 + kw
            wk = combb[:, k * C_IN:(k + 1) * C_IN]
            xs = xv[:, kh * W + kw: kh * W + kw + HOW]
            acc = acc + jax.lax.dot_general(
                wk, xs, (((1,), (0,)), ((), ())),
                preferred_element_type=jnp.float32)
    out_ref[0] = acc + beff


def kernel(x, task, routing_vector, W_route, b_route, emb, conv_w, conv_b):
    B, C_IN, H, W = x.shape
    E, C_OUT = conv_b.shape
    HO, WO = H - 2, W - 2
    R = routing_vector.shape[1]

    cw3 = conv_w.transpose(0, 1, 3, 4, 2).reshape(E, C_OUT, 9 * C_IN)
    br2 = b_route.reshape(1, -1)
    rv3 = routing_vector.reshape(B, 1, R)
    xf = jnp.pad(x.reshape(B, C_IN, H * W),
                 ((0, 0), (0, 0), (0, 2))).astype(jnp.bfloat16)

    wide = pl.pallas_call(
        _body,
        grid=(B,),
        in_specs=[
            pl.BlockSpec((1, C_IN, H * W + 2), lambda b: (b, 0, 0)),
            pl.BlockSpec((1, 1, R), lambda b: (b, 0, 0)),
            pl.BlockSpec(W_route.shape, lambda b: (0, 0)),
            pl.BlockSpec(br2.shape, lambda b: (0, 0)),
            pl.BlockSpec(emb.shape, lambda b: (0, 0)),
            pl.BlockSpec(cw3.shape, lambda b: (0, 0, 0)),
            pl.BlockSpec(conv_b.shape, lambda b: (0, 0)),
        ],
        out_specs=pl.BlockSpec((1, C_OUT, HO * W), lambda b: (b, 0, 0)),
        out_shape=jax.ShapeDtypeStruct((B, C_OUT, HO * W), jnp.float32),
    )(xf, rv3, W_route, br2, emb, cw3, conv_b)
    return wide.reshape(B, C_OUT, HO, W)[:, :, :, :WO]


# speedup vs baseline: 1.2864x; 1.2864x over previous
import jax
import jax.numpy as jnp
from jax.experimental import pallas as pl


def _body(x_ref, rv_ref, wr_ref, br_ref, emb_ref, cw_ref, cb_ref, out_ref):
    E, C_OUT, KC = cw_ref.shape
    C_IN = KC // 9
    B = x_ref.shape[0]
    HW = x_ref.shape[2]
    HOW = out_ref.shape[2]
    W = (HW - HOW) // 2

    rv = rv_ref[...]
    r = jax.lax.dot_general(rv, wr_ref[...],
                            (((1,), (1,)), ((), ())),
                            preferred_element_type=jnp.float32)
    r = r + br_ref[...]

    rn = r / (jnp.sqrt(jnp.sum(r * r, axis=-1, keepdims=True)) + 1e-8)
    emb = emb_ref[...]
    en = emb / (jnp.sqrt(jnp.sum(emb * emb, axis=-1, keepdims=True)) + 1e-8)
    sim = jax.lax.dot_general(en, rn, (((1,), (1,)), ((), ())),
                              preferred_element_type=jnp.float32)

    m = jnp.max(sim, axis=0, keepdims=True)
    ex = jnp.exp(sim - m)
    wts = ex / jnp.sum(ex, axis=0, keepdims=True)
    wts = wts / jnp.sum(wts, axis=0, keepdims=True)

    beff = jax.lax.dot_general(cb_ref[...], wts, (((0,), (0,)), ((), ())),
                               preferred_element_type=jnp.float32)

    cw = cw_ref[...]
    for b in range(B):
        comb = jnp.sum(cw.astype(jnp.float32) * wts[:, b:b + 1].reshape(E, 1, 1),
                       axis=0)
        combb = comb.astype(jnp.bfloat16)
        xv = x_ref[b].astype(jnp.bfloat16)
        acc = jnp.zeros((C_OUT, HOW), dtype=jnp.float32)
        for kh in range(3):
            for kw in range(3):
                k = kh * 3 + kw
                wk = combb[:, k * C_IN:(k + 1) * C_IN]
                start = kh * W + kw
                if start + HOW <= HW:
                    xs = xv[:, start:start + HOW]
                else:
                    xs = jnp.pad(xv[:, start:HW],
                                 ((0, 0), (0, start + HOW - HW)))
                acc = acc + jax.lax.dot_general(
                    wk, xs, (((1,), (0,)), ((), ())),
                    preferred_element_type=jnp.float32)
        out_ref[b] = acc + beff[:, b:b + 1]


def kernel(x, task, routing_vector, W_route, b_route, emb, conv_w, conv_b):
    B, C_IN, H, W = x.shape
    E, C_OUT = conv_b.shape
    HO, WO = H - 2, W - 2
    R = routing_vector.shape[1]

    cw3 = conv_w.transpose(0, 1, 3, 4, 2).reshape(E, C_OUT, 9 * C_IN)
    cw3 = cw3.astype(jnp.bfloat16)
    br2 = b_route.reshape(1, -1)
    xf = x.reshape(B, C_IN, H * W)

    wide = pl.pallas_call(
        _body,
        in_specs=[
            pl.BlockSpec(xf.shape, lambda: (0, 0, 0)),
            pl.BlockSpec(routing_vector.shape, lambda: (0, 0)),
            pl.BlockSpec(W_route.shape, lambda: (0, 0)),
            pl.BlockSpec(br2.shape, lambda: (0, 0)),
            pl.BlockSpec(emb.shape, lambda: (0, 0)),
            pl.BlockSpec(cw3.shape, lambda: (0, 0, 0)),
            pl.BlockSpec(conv_b.shape, lambda: (0, 0)),
        ],
        out_specs=pl.BlockSpec((B, C_OUT, HO * W), lambda: (0, 0, 0)),
        out_shape=jax.ShapeDtypeStruct((B, C_OUT, HO * W), jnp.float32),
    )(xf, routing_vector, W_route, br2, emb, cw3, conv_b)
    return wide.reshape(B, C_OUT, HO, W)[:, :, :, :WO]
